# baseline (speedup 1.0000x reference)
import jax
import jax.numpy as jnp
from jax.experimental import pallas as pl
from jax.experimental.pallas import tpu as pltpu

D_DEPTH = 16
L_WIDTH = 1024
P_PAR = 16
H = 64
E = 64
BATCH = 32
NCORE = 2
BG = BATCH // NCORE
TL = 512
NT = L_WIDTH // TL


def _dag_body(emb_ref, table_ref, w1p_ref, w1e_ref, b1_ref, w2_ref, b2_ref,
              woutm_ref, boutm_ref, woutr_ref, boutr_ref,
              outm_ref, outr_ref, slab_ref):
    d = pl.program_id(1)
    t = pl.program_id(2)

    @pl.when(jnp.logical_and(d == 0, t == 0))
    def _root_out():
        outr_ref[:] = (jnp.sum(emb_ref[:] * woutr_ref[:], axis=1,
                               keepdims=True) + boutr_ref[:])

    r = (d + 1) % 2
    w = d % 2

    base = t * TL
    nxt = ((t + 1) % NT) * TL

    @pl.when(d == 0)
    def _seed():
        seed = jnp.broadcast_to((emb_ref[:] * 0.0625)[:, None, :],
                                (BG, TL, H))
        slab_ref[r, :, pl.ds(base, TL), :] = seed
        slab_ref[r, :, pl.ds(nxt, 16), :] = seed[:, :16, :]

    x = slab_ref[r, :, pl.ds(base, TL), :]
    x16 = slab_ref[r, :, pl.ds(nxt, 16), :]
    s = jnp.concatenate([x, x16], axis=1)
    for k in (1, 2, 4, 8):
        s = s[:, :-k, :] + s[:, k:, :]
    pv = s[:, :TL, :]

    np_tile = jax.lax.dot_general(
        table_ref[:], w1e_ref[:],
        dimension_numbers=(((1,), (1,)), ((), ())),
        preferred_element_type=jnp.float32) + b1_ref[:]
    pv2 = pv.reshape(BG * TL, H)
    h1 = jax.lax.dot_general(
        pv2, w1p_ref[:],
        dimension_numbers=(((1,), (1,)), ((), ())),
        preferred_element_type=jnp.float32).reshape(BG, TL, H)
    h1 = h1 + np_tile[None]
    a = jnp.maximum(h1, 0.0).reshape(BG * TL, H)
    h2 = jax.lax.dot_general(
        a, w2_ref[:],
        dimension_numbers=(((1,), (1,)), ((), ())),
        preferred_element_type=jnp.float32).reshape(BG, TL, H)
    cur = pv + h2 + b2_ref[:][None]

    slab_ref[w, :, pl.ds(base, TL), :] = cur
    outm_ref[0] = (jnp.sum(cur * woutm_ref[:][None], axis=2)
                   + boutm_ref[0, 0][None, :])


def _run(emb, table, w1p, w1e, b1, w2, b2, woutm, boutm, woutr, boutr):
    grid = (NCORE, D_DEPTH, NT)
    outm, outr = pl.pallas_call(
        _dag_body,
        grid=grid,
        in_specs=[
            pl.BlockSpec((BG, H), lambda g, d, t: (g, 0)),
            pl.BlockSpec((TL, E), lambda g, d, t: (d * NT + t, 0)),
            pl.BlockSpec((H, H), lambda g, d, t: (0, 0)),
            pl.BlockSpec((H, E), lambda g, d, t: (0, 0)),
            pl.BlockSpec((1, H), lambda g, d, t: (0, 0)),
            pl.BlockSpec((H, H), lambda g, d, t: (0, 0)),
            pl.BlockSpec((1, H), lambda g, d, t: (0, 0)),
            pl.BlockSpec((TL, H), lambda g, d, t: (d * NT + t, 0)),
            pl.BlockSpec((1, 1, TL),
                         lambda g, d, t: (d * NT + t, 0, 0)),
            pl.BlockSpec((1, H), lambda g, d, t: (0, 0)),
            pl.BlockSpec((1, 1), lambda g, d, t: (0, 0)),
        ],
        out_specs=[
            pl.BlockSpec((1, BG, TL), lambda g, d, t: (d, g, t)),
            pl.BlockSpec((BG, 1), lambda g, d, t: (g, 0)),
        ],
        out_shape=[
            jax.ShapeDtypeStruct((D_DEPTH, BATCH, L_WIDTH), jnp.float32),
            jax.ShapeDtypeStruct((BATCH, 1), jnp.float32),
        ],
        scratch_shapes=[pltpu.VMEM((2, BG, L_WIDTH, H), jnp.float32)],
        compiler_params=pltpu.CompilerParams(
            dimension_semantics=("parallel", "arbitrary", "arbitrary")),
    )(emb, table, w1p, w1e, b1, w2, b2, woutm, boutm, woutr, boutr)
    return outm, outr


def kernel(embedding, emb_table, W1, b1, W2, b2, Wout, bout):
    table = emb_table[2:2 + D_DEPTH * L_WIDTH]
    w1p = W1[:, :H]
    w1e = W1[:, H:]
    woutm = Wout[0, 1:, :]
    boutm = bout[0, 1:].reshape(D_DEPTH * NT, 1, TL)
    woutr = Wout[0, 0:1, :]
    boutr = bout[:, 0:1]
    outm, outr = _run(embedding, table, w1p, w1e, b1.reshape(1, H), W2,
                      b2.reshape(1, H), woutm, boutm, woutr, boutr)
    out_main = outm.transpose(1, 0, 2).reshape(BATCH, D_DEPTH * L_WIDTH)
    return jnp.concatenate([outr, out_main], axis=1)


# speedup vs baseline: 1.2565x; 1.2565x over previous
import jax
import jax.numpy as jnp
from jax.experimental import pallas as pl
from jax.experimental.pallas import tpu as pltpu

D_DEPTH = 16
L_WIDTH = 1024
P_PAR = 16
H = 64
E = 64
BATCH = 32
NCORE = 1
BG = BATCH // NCORE
TL = 512
NT = L_WIDTH // TL


def _dag_body(emb_ref, table_ref, w1p_ref, w1e_ref, b1_ref, w2_ref, b2_ref,
              woutm_ref, boutm_ref, woutr_ref, boutr_ref,
              outm_ref, outr_ref, slab_ref):
    d = pl.program_id(1)
    t = pl.program_id(2)

    @pl.when(jnp.logical_and(d == 0, t == 0))
    def _root_out():
        outr_ref[:] = (jnp.sum(emb_ref[:] * woutr_ref[:], axis=1,
                               keepdims=True) + boutr_ref[:])

    r = (d + 1) % 2
    w = d % 2

    base = t * TL
    nxt = ((t + 1) % NT) * TL

    @pl.when(d == 0)
    def _seed():
        seed = jnp.broadcast_to((emb_ref[:] * 0.0625)[:, None, :],
                                (BG, TL, H))
        slab_ref[r, :, pl.ds(base, TL), :] = seed
        slab_ref[r, :, pl.ds(nxt, 16), :] = seed[:, :16, :]

    x = slab_ref[r, :, pl.ds(base, TL), :]
    x16 = slab_ref[r, :, pl.ds(nxt, 16), :]
    s = jnp.concatenate([x, x16], axis=1)
    for k in (1, 2, 4, 8):
        s = s[:, :-k, :] + s[:, k:, :]
    pv = s[:, :TL, :]

    np_tile = jax.lax.dot_general(
        table_ref[:], w1e_ref[:],
        dimension_numbers=(((1,), (1,)), ((), ())),
        preferred_element_type=jnp.float32) + b1_ref[:]
    pv2 = pv.reshape(BG * TL, H)
    h1 = jax.lax.dot_general(
        pv2, w1p_ref[:],
        dimension_numbers=(((1,), (1,)), ((), ())),
        preferred_element_type=jnp.float32).reshape(BG, TL, H)
    h1 = h1 + np_tile[None]
    a = jnp.maximum(h1, 0.0).reshape(BG * TL, H)
    h2 = jax.lax.dot_general(
        a, w2_ref[:],
        dimension_numbers=(((1,), (1,)), ((), ())),
        preferred_element_type=jnp.float32).reshape(BG, TL, H)
    cur = pv + h2 + b2_ref[:][None]

    slab_ref[w, :, pl.ds(base, TL), :] = cur
    outm_ref[0] = (jnp.sum(cur * woutm_ref[:][None], axis=2)
                   + boutm_ref[0, 0][None, :])


def _run(emb, table, w1p, w1e, b1, w2, b2, woutm, boutm, woutr, boutr):
    grid = (NCORE, D_DEPTH, NT)
    outm, outr = pl.pallas_call(
        _dag_body,
        grid=grid,
        in_specs=[
            pl.BlockSpec((BG, H), lambda g, d, t: (g, 0)),
            pl.BlockSpec((TL, E), lambda g, d, t: (d * NT + t, 0)),
            pl.BlockSpec((H, H), lambda g, d, t: (0, 0)),
            pl.BlockSpec((H, E), lambda g, d, t: (0, 0)),
            pl.BlockSpec((1, H), lambda g, d, t: (0, 0)),
            pl.BlockSpec((H, H), lambda g, d, t: (0, 0)),
            pl.BlockSpec((1, H), lambda g, d, t: (0, 0)),
            pl.BlockSpec((TL, H), lambda g, d, t: (d * NT + t, 0)),
            pl.BlockSpec((1, 1, TL),
                         lambda g, d, t: (d * NT + t, 0, 0)),
            pl.BlockSpec((1, H), lambda g, d, t: (0, 0)),
            pl.BlockSpec((1, 1), lambda g, d, t: (0, 0)),
        ],
        out_specs=[
            pl.BlockSpec((1, BG, TL), lambda g, d, t: (d, g, t)),
            pl.BlockSpec((BG, 1), lambda g, d, t: (g, 0)),
        ],
        out_shape=[
            jax.ShapeDtypeStruct((D_DEPTH, BATCH, L_WIDTH), jnp.float32),
            jax.ShapeDtypeStruct((BATCH, 1), jnp.float32),
        ],
        scratch_shapes=[pltpu.VMEM((2, BG, L_WIDTH, H), jnp.float32)],
        compiler_params=pltpu.CompilerParams(
            dimension_semantics=("parallel", "arbitrary", "arbitrary")),
    )(emb, table, w1p, w1e, b1, w2, b2, woutm, boutm, woutr, boutr)
    return outm, outr


def kernel(embedding, emb_table, W1, b1, W2, b2, Wout, bout):
    table = emb_table[2:2 + D_DEPTH * L_WIDTH]
    w1p = W1[:, :H]
    w1e = W1[:, H:]
    woutm = Wout[0, 1:, :]
    boutm = bout[0, 1:].reshape(D_DEPTH * NT, 1, TL)
    woutr = Wout[0, 0:1, :]
    boutr = bout[:, 0:1]
    outm, outr = _run(embedding, table, w1p, w1e, b1.reshape(1, H), W2,
                      b2.reshape(1, H), woutm, boutm, woutr, boutr)
    out_main = outm.transpose(1, 0, 2).reshape(BATCH, D_DEPTH * L_WIDTH)
    return jnp.concatenate([outr, out_main], axis=1)


# speedup vs baseline: 1.9098x; 1.5200x over previous
import jax
import jax.numpy as jnp
from jax.experimental import pallas as pl
from jax.experimental.pallas import tpu as pltpu

D_DEPTH = 16
L_WIDTH = 1024
P_PAR = 16
H = 64
E = 64
BATCH = 32
NCORE = 1
BG = BATCH // NCORE
TL = 512
NT = L_WIDTH // TL


def _dag_body(emb_ref, table_ref, w1p_ref, w1e_ref, b1_ref, w2_ref, b2_ref,
              woutm_ref, boutm_ref, woutr_ref, boutr_ref,
              outm_ref, outr_ref, slab_ref):
    d = pl.program_id(1)
    t = pl.program_id(2)

    @pl.when(jnp.logical_and(d == 0, t == 0))
    def _root_out():
        outr_ref[:] = (jnp.sum(emb_ref[:] * woutr_ref[:], axis=1,
                               keepdims=True) + boutr_ref[:])

    r = (d + 1) % 2
    w = d % 2

    base = t * TL
    nxt = ((t + 1) % NT) * TL

    @pl.when(d == 0)
    def _seed():
        seed = jnp.broadcast_to((emb_ref[:] * 0.0625)[:, None, :],
                                (BG, TL, H))
        slab_ref[r, :, pl.ds(base, TL), :] = seed
        slab_ref[r, :, pl.ds(nxt, 16), :] = seed[:, :16, :]

    x = slab_ref[r, :, pl.ds(base, TL), :]
    x16 = slab_ref[r, :, pl.ds(nxt, 16), :]
    s = jnp.concatenate([x, x16], axis=1)
    for k in (1, 2, 4, 8):
        s = s[:, :-k, :] + s[:, k:, :]
    pv = s[:, :TL, :]

    np_tile = jax.lax.dot_general(
        table_ref[:], w1e_ref[:],
        dimension_numbers=(((1,), (1,)), ((), ())),
        preferred_element_type=jnp.float32) + b1_ref[:]
    pv2 = pv.reshape(BG * TL, H)
    h1 = jax.lax.dot_general(
        pv2, w1p_ref[:],
        dimension_numbers=(((1,), (1,)), ((), ())),
        preferred_element_type=jnp.float32).reshape(BG, TL, H)
    h1 = h1 + np_tile[None]
    a = jnp.maximum(h1, 0.0).reshape(BG * TL, H)
    h2 = jax.lax.dot_general(
        a, w2_ref[:],
        dimension_numbers=(((1,), (1,)), ((), ())),
        preferred_element_type=jnp.float32).reshape(BG, TL, H)
    cur = pv + h2 + b2_ref[:][None]

    slab_ref[w, :, pl.ds(base, TL), :] = cur
    p = cur * woutm_ref[:][None]
    ones = jnp.ones((1, 1, H), jnp.float32)
    red = jax.lax.dot_general(
        jnp.broadcast_to(ones, (BG, 1, H)), p,
        dimension_numbers=(((2,), (2,)), ((0,), (0,))),
        preferred_element_type=jnp.float32)
    outm_ref[0] = red[:, 0, :] + boutm_ref[0, 0][None, :]


def _run(emb, table, w1p, w1e, b1, w2, b2, woutm, boutm, woutr, boutr):
    grid = (NCORE, D_DEPTH, NT)
    outm, outr = pl.pallas_call(
        _dag_body,
        grid=grid,
        in_specs=[
            pl.BlockSpec((BG, H), lambda g, d, t: (g, 0)),
            pl.BlockSpec((TL, E), lambda g, d, t: (d * NT + t, 0)),
            pl.BlockSpec((H, H), lambda g, d, t: (0, 0)),
            pl.BlockSpec((H, E), lambda g, d, t: (0, 0)),
            pl.BlockSpec((1, H), lambda g, d, t: (0, 0)),
            pl.BlockSpec((H, H), lambda g, d, t: (0, 0)),
            pl.BlockSpec((1, H), lambda g, d, t: (0, 0)),
            pl.BlockSpec((TL, H), lambda g, d, t: (d * NT + t, 0)),
            pl.BlockSpec((1, 1, TL),
                         lambda g, d, t: (d * NT + t, 0, 0)),
            pl.BlockSpec((1, H), lambda g, d, t: (0, 0)),
            pl.BlockSpec((1, 1), lambda g, d, t: (0, 0)),
        ],
        out_specs=[
            pl.BlockSpec((1, BG, TL), lambda g, d, t: (d, g, t)),
            pl.BlockSpec((BG, 1), lambda g, d, t: (g, 0)),
        ],
        out_shape=[
            jax.ShapeDtypeStruct((D_DEPTH, BATCH, L_WIDTH), jnp.float32),
            jax.ShapeDtypeStruct((BATCH, 1), jnp.float32),
        ],
        scratch_shapes=[pltpu.VMEM((2, BG, L_WIDTH, H), jnp.float32)],
        compiler_params=pltpu.CompilerParams(
            dimension_semantics=("parallel", "arbitrary", "arbitrary")),
    )(emb, table, w1p, w1e, b1, w2, b2, woutm, boutm, woutr, boutr)
    return outm, outr


def kernel(embedding, emb_table, W1, b1, W2, b2, Wout, bout):
    table = emb_table[2:2 + D_DEPTH * L_WIDTH]
    w1p = W1[:, :H]
    w1e = W1[:, H:]
    woutm = Wout[0, 1:, :]
    boutm = bout[0, 1:].reshape(D_DEPTH * NT, 1, TL)
    woutr = Wout[0, 0:1, :]
    boutr = bout[:, 0:1]
    outm, outr = _run(embedding, table, w1p, w1e, b1.reshape(1, H), W2,
                      b2.reshape(1, H), woutm, boutm, woutr, boutr)
    out_main = outm.transpose(1, 0, 2).reshape(BATCH, D_DEPTH * L_WIDTH)
    return jnp.concatenate([outr, out_main], axis=1)


# speedup vs baseline: 2.8463x; 1.4903x over previous
import jax
import jax.numpy as jnp
from jax.experimental import pallas as pl
from jax.experimental.pallas import tpu as pltpu

D_DEPTH = 16
L_WIDTH = 1024
P_PAR = 16
H = 64
E = 64
BATCH = 32
BG = BATCH // 2
C = 2 * H
TL = 512
NT = L_WIDTH // TL


def _blockdiag(w):
    z = jnp.zeros((C, C), w.dtype)
    return z.at[:H, :H].set(w).at[H:, H:].set(w)


def kernel(embedding, emb_table, W1, b1, W2, b2, Wout, bout):
    table = emb_table[2:2 + D_DEPTH * L_WIDTH]
    table2 = jnp.concatenate([table, table], axis=1)
    w1bd = _blockdiag(W1[:, H:])
    w1pbd = _blockdiag(W1[:, :H])
    w2bd = _blockdiag(W2)
    emb2 = embedding.reshape(BG, C)
    woutm = Wout[0, 1:, :]
    woutm2 = jnp.concatenate([woutm, woutm], axis=1)
    boutm = bout[0, 1:].reshape(D_DEPTH * NT, 1, TL)
    woutr2 = jnp.concatenate([Wout[0, 0:1, :]] * 2, axis=1)
    boutr = jnp.broadcast_to(bout[:, 0:1], (BG, 2))
    b1p = jnp.concatenate([b1, b1]).reshape(1, C)
    b2p = jnp.concatenate([b2, b2]).reshape(1, C)

    outm, outr = _run(emb2, table2, w1pbd, w1bd, b1p, w2bd, b2p,
                               woutm2, boutm, woutr2, boutr)
    out_main = (outm.transpose(1, 2, 0, 3)
                .reshape(BATCH, D_DEPTH * L_WIDTH))
    out_root = outr.reshape(BATCH, 1)
    return jnp.concatenate([out_root, out_main], axis=1)


def _run(emb2, table2, w1pbd, w1ebd, b1p, w2bd, b2p,
                  woutm2, boutm, woutr2, boutr):
    grid = (D_DEPTH, NT)
    outm, outr = pl.pallas_call(
        _dag_body,
        grid=grid,
        in_specs=[
            pl.BlockSpec((BG, C), lambda d, t: (0, 0)),
            pl.BlockSpec((TL, C), lambda d, t: (d * NT + t, 0)),
            pl.BlockSpec((C, C), lambda d, t: (0, 0)),
            pl.BlockSpec((C, C), lambda d, t: (0, 0)),
            pl.BlockSpec((1, C), lambda d, t: (0, 0)),
            pl.BlockSpec((C, C), lambda d, t: (0, 0)),
            pl.BlockSpec((1, C), lambda d, t: (0, 0)),
            pl.BlockSpec((TL, C), lambda d, t: (d * NT + t, 0)),
            pl.BlockSpec((1, 1, TL),
                         lambda d, t: (d * NT + t, 0, 0)),
            pl.BlockSpec((1, C), lambda d, t: (0, 0)),
            pl.BlockSpec((BG, 2), lambda d, t: (0, 0)),
        ],
        out_specs=[
            pl.BlockSpec((1, BG, 2, TL), lambda d, t: (d, 0, 0, t)),
            pl.BlockSpec((BG, 2), lambda d, t: (0, 0)),
        ],
        out_shape=[
            jax.ShapeDtypeStruct((D_DEPTH, BG, 2, L_WIDTH), jnp.float32),
            jax.ShapeDtypeStruct((BG, 2), jnp.float32),
        ],
        scratch_shapes=[pltpu.VMEM((2, BG, L_WIDTH, C), jnp.float32)],
    )(emb2, table2, w1pbd, w1ebd, b1p, w2bd, b2p, woutm2, boutm,
      woutr2, boutr)
    return outm, outr


def _dag_body(emb_ref, table_ref, w1p_ref, w1e_ref, b1_ref, w2_ref, b2_ref,
               woutm_ref, boutm_ref, woutr_ref, boutr_ref,
               outm_ref, outr_ref, slab_ref):
    d = pl.program_id(0)
    t = pl.program_id(1)

    sel = (jax.lax.broadcasted_iota(jnp.int32, (2, C), 1) // H ==
           jax.lax.broadcasted_iota(jnp.int32, (2, C), 0)).astype(jnp.float32)

    @pl.when(jnp.logical_and(d == 0, t == 0))
    def _root_out():
        prod = emb_ref[:] * woutr_ref[:]
        outr_ref[:] = jax.lax.dot_general(
            prod, sel,
            dimension_numbers=(((1,), (1,)), ((), ())),
            preferred_element_type=jnp.float32) + boutr_ref[:]

    r = (d + 1) % 2
    w = d % 2
    base = t * TL
    nxt = ((t + 1) % NT) * TL

    @pl.when(d == 0)
    def _seed():
        seed = jnp.broadcast_to((emb_ref[:] * 0.0625)[:, None, :],
                                (BG, TL, C))
        slab_ref[r, :, pl.ds(base, TL), :] = seed
        slab_ref[r, :, pl.ds(nxt, 16), :] = seed[:, :16, :]

    x = slab_ref[r, :, pl.ds(base, TL), :]
    x16 = slab_ref[r, :, pl.ds(nxt, 16), :]
    s = jnp.concatenate([x, x16], axis=1)
    for k in (1, 2, 4, 8):
        s = s[:, :-k, :] + s[:, k:, :]
    pv = s[:, :TL, :]

    np_tile = jax.lax.dot_general(
        table_ref[:], w1e_ref[:],
        dimension_numbers=(((1,), (1,)), ((), ())),
        preferred_element_type=jnp.float32) + b1_ref[:]
    pv2 = pv.reshape(BG * TL, C)
    h1 = jax.lax.dot_general(
        pv2, w1p_ref[:],
        dimension_numbers=(((1,), (1,)), ((), ())),
        preferred_element_type=jnp.float32).reshape(BG, TL, C)
    h1 = h1 + np_tile[None]
    a = jnp.maximum(h1, 0.0).reshape(BG * TL, C)
    h2 = jax.lax.dot_general(
        a, w2_ref[:],
        dimension_numbers=(((1,), (1,)), ((), ())),
        preferred_element_type=jnp.float32).reshape(BG, TL, C)
    cur = pv + h2 + b2_ref[:][None]

    slab_ref[w, :, pl.ds(base, TL), :] = cur
    p = cur * woutm_ref[:][None]
    red = jax.lax.dot_general(
        jnp.broadcast_to(sel[None], (BG, 2, C)), p,
        dimension_numbers=(((2,), (2,)), ((0,), (0,))),
        preferred_element_type=jnp.float32)
    outm_ref[0] = red + boutm_ref[0, 0][None, None, :]


# speedup vs baseline: 2.8480x; 1.0006x over previous
import jax
import jax.numpy as jnp
from jax.experimental import pallas as pl
from jax.experimental.pallas import tpu as pltpu

D_DEPTH = 16
L_WIDTH = 1024
P_PAR = 16
H = 64
E = 64
BATCH = 32
BG = BATCH // 2
C = 2 * H
TL = 512
NT = L_WIDTH // TL


def _blockdiag(w):
    z = jnp.zeros((C, C), w.dtype)
    return z.at[:H, :H].set(w).at[H:, H:].set(w)


def kernel(embedding, emb_table, W1, b1, W2, b2, Wout, bout):
    table = emb_table[2:2 + D_DEPTH * L_WIDTH]
    table2 = jnp.concatenate([table, table], axis=1)
    w1bd = _blockdiag(W1[:, H:])
    w1pbd = _blockdiag(W1[:, :H])
    w2bd = _blockdiag(W2)
    emb2 = embedding.reshape(BG, C)
    woutm = Wout[0, 1:, :]
    woutm2 = jnp.concatenate([woutm, woutm], axis=1)
    boutm = bout[0, 1:].reshape(D_DEPTH * NT, 1, TL)
    woutr2 = jnp.concatenate([Wout[0, 0:1, :]] * 2, axis=1)
    boutr = jnp.broadcast_to(bout[:, 0:1], (BG, 2))
    b1p = jnp.concatenate([b1, b1]).reshape(1, C)
    b2p = jnp.concatenate([b2, b2]).reshape(1, C)

    outm, outr = _run(emb2, table2, w1pbd, w1bd, b1p, w2bd, b2p,
                               woutm2, boutm, woutr2, boutr)
    out_main = (outm.transpose(1, 2, 0, 3)
                .reshape(BATCH, D_DEPTH * L_WIDTH))
    out_root = outr.reshape(BATCH, 1)
    return jnp.concatenate([out_root, out_main], axis=1)


def _run(emb2, table2, w1pbd, w1ebd, b1p, w2bd, b2p,
                  woutm2, boutm, woutr2, boutr):
    grid = (D_DEPTH, NT)
    outm, outr = pl.pallas_call(
        _dag_body,
        grid=grid,
        in_specs=[
            pl.BlockSpec((BG, C), lambda d, t: (0, 0)),
            pl.BlockSpec((TL, C), lambda d, t: (d * NT + t, 0)),
            pl.BlockSpec((C, C), lambda d, t: (0, 0)),
            pl.BlockSpec((C, C), lambda d, t: (0, 0)),
            pl.BlockSpec((1, C), lambda d, t: (0, 0)),
            pl.BlockSpec((C, C), lambda d, t: (0, 0)),
            pl.BlockSpec((1, C), lambda d, t: (0, 0)),
            pl.BlockSpec((TL, C), lambda d, t: (d * NT + t, 0)),
            pl.BlockSpec((1, 1, TL),
                         lambda d, t: (d * NT + t, 0, 0)),
            pl.BlockSpec((1, C), lambda d, t: (0, 0)),
            pl.BlockSpec((BG, 2), lambda d, t: (0, 0)),
        ],
        out_specs=[
            pl.BlockSpec((1, BG, 2, TL), lambda d, t: (d, 0, 0, t)),
            pl.BlockSpec((BG, 2), lambda d, t: (0, 0)),
        ],
        out_shape=[
            jax.ShapeDtypeStruct((D_DEPTH, BG, 2, L_WIDTH), jnp.float32),
            jax.ShapeDtypeStruct((BG, 2), jnp.float32),
        ],
        scratch_shapes=[pltpu.VMEM((2, BG, L_WIDTH, C), jnp.float32)],
    )(emb2, table2, w1pbd, w1ebd, b1p, w2bd, b2p, woutm2, boutm,
      woutr2, boutr)
    return outm, outr


def _dag_body(emb_ref, table_ref, w1p_ref, w1e_ref, b1_ref, w2_ref, b2_ref,
               woutm_ref, boutm_ref, woutr_ref, boutr_ref,
               outm_ref, outr_ref, slab_ref):
    d = pl.program_id(0)
    t = pl.program_id(1)

    sel = (jax.lax.broadcasted_iota(jnp.int32, (2, C), 1) // H ==
           jax.lax.broadcasted_iota(jnp.int32, (2, C), 0)).astype(jnp.float32)

    @pl.when(jnp.logical_and(d == 0, t == 0))
    def _root_out():
        prod = emb_ref[:] * woutr_ref[:]
        outr_ref[:] = jax.lax.dot_general(
            prod, sel,
            dimension_numbers=(((1,), (1,)), ((), ())),
            preferred_element_type=jnp.float32) + boutr_ref[:]

    r = (d + 1) % 2
    w = d % 2
    base = t * TL
    nxt = ((t + 1) % NT) * TL

    @pl.when(d == 0)
    def _seed():
        seed = jnp.broadcast_to((emb_ref[:] * 0.0625)[:, None, :],
                                (BG, TL, C))
        slab_ref[r, :, pl.ds(base, TL), :] = seed
        slab_ref[r, :, pl.ds(nxt, 16), :] = seed[:, :16, :]

    x = slab_ref[r, :, pl.ds(base, TL), :]
    x16 = slab_ref[r, :, pl.ds(nxt, 16), :]
    s = jnp.concatenate([x, x16], axis=1)
    ri = jax.lax.broadcasted_iota(jnp.int32, (128, 144), 0)
    ci = jax.lax.broadcasted_iota(jnp.int32, (128, 144), 1)
    band = jnp.logical_and(ci >= ri, ci < ri + 16).astype(jnp.float32)
    bandb = jnp.broadcast_to(band[None], (BG, 128, 144))
    parts = []
    for u in range(TL // 128):
        su = s[:, 128 * u:128 * u + 144, :]
        parts.append(jax.lax.dot_general(
            bandb, su,
            dimension_numbers=(((2,), (1,)), ((0,), (0,))),
            preferred_element_type=jnp.float32))
    pv = jnp.concatenate(parts, axis=1)

    np_tile = jax.lax.dot_general(
        table_ref[:], w1e_ref[:],
        dimension_numbers=(((1,), (1,)), ((), ())),
        preferred_element_type=jnp.float32) + b1_ref[:]
    pv2 = pv.reshape(BG * TL, C)
    h1 = jax.lax.dot_general(
        pv2, w1p_ref[:],
        dimension_numbers=(((1,), (1,)), ((), ())),
        preferred_element_type=jnp.float32).reshape(BG, TL, C)
    h1 = h1 + np_tile[None]
    a = jnp.maximum(h1, 0.0).reshape(BG * TL, C)
    h2 = jax.lax.dot_general(
        a, w2_ref[:],
        dimension_numbers=(((1,), (1,)), ((), ())),
        preferred_element_type=jnp.float32).reshape(BG, TL, C)
    cur = pv + h2 + b2_ref[:][None]

    slab_ref[w, :, pl.ds(base, TL), :] = cur
    p = cur * woutm_ref[:][None]
    red = jax.lax.dot_general(
        jnp.broadcast_to(sel[None], (BG, 2, C)), p,
        dimension_numbers=(((2,), (2,)), ((0,), (0,))),
        preferred_element_type=jnp.float32)
    outm_ref[0] = red + boutm_ref[0, 0][None, None, :]


# speedup vs baseline: 2.9074x; 1.0209x over previous
import jax
import jax.numpy as jnp
from jax.experimental import pallas as pl
from jax.experimental.pallas import tpu as pltpu

D_DEPTH = 16
L_WIDTH = 1024
P_PAR = 16
H = 64
E = 64
BATCH = 32
BG = BATCH // 2
C = 2 * H
TL = 1024
NT = L_WIDTH // TL


def _blockdiag(w):
    z = jnp.zeros((C, C), w.dtype)
    return z.at[:H, :H].set(w).at[H:, H:].set(w)


def kernel(embedding, emb_table, W1, b1, W2, b2, Wout, bout):
    table = emb_table[2:2 + D_DEPTH * L_WIDTH]
    table2 = jnp.concatenate([table, table], axis=1)
    w1bd = _blockdiag(W1[:, H:])
    w1pbd = _blockdiag(W1[:, :H])
    w2bd = _blockdiag(W2)
    emb2 = embedding.reshape(BG, C)
    woutm = Wout[0, 1:, :]
    woutm2 = jnp.concatenate([woutm, woutm], axis=1)
    boutm = bout[0, 1:].reshape(D_DEPTH * NT, 1, TL)
    woutr2 = jnp.concatenate([Wout[0, 0:1, :]] * 2, axis=1)
    boutr = jnp.broadcast_to(bout[:, 0:1], (BG, 2))
    b1p = jnp.concatenate([b1, b1]).reshape(1, C)
    b2p = jnp.concatenate([b2, b2]).reshape(1, C)

    outm, outr = _run(emb2, table2, w1pbd, w1bd, b1p, w2bd, b2p,
                               woutm2, boutm, woutr2, boutr)
    out_main = (outm.transpose(1, 2, 0, 3)
                .reshape(BATCH, D_DEPTH * L_WIDTH))
    out_root = outr.reshape(BATCH, 1)
    return jnp.concatenate([out_root, out_main], axis=1)


def _run(emb2, table2, w1pbd, w1ebd, b1p, w2bd, b2p,
                  woutm2, boutm, woutr2, boutr):
    grid = (D_DEPTH, NT)
    outm, outr = pl.pallas_call(
        _dag_body,
        grid=grid,
        in_specs=[
            pl.BlockSpec((BG, C), lambda d, t: (0, 0)),
            pl.BlockSpec((TL, C), lambda d, t: (d * NT + t, 0)),
            pl.BlockSpec((C, C), lambda d, t: (0, 0)),
            pl.BlockSpec((C, C), lambda d, t: (0, 0)),
            pl.BlockSpec((1, C), lambda d, t: (0, 0)),
            pl.BlockSpec((C, C), lambda d, t: (0, 0)),
            pl.BlockSpec((1, C), lambda d, t: (0, 0)),
            pl.BlockSpec((TL, C), lambda d, t: (d * NT + t, 0)),
            pl.BlockSpec((1, 1, TL),
                         lambda d, t: (d * NT + t, 0, 0)),
            pl.BlockSpec((1, C), lambda d, t: (0, 0)),
            pl.BlockSpec((BG, 2), lambda d, t: (0, 0)),
        ],
        out_specs=[
            pl.BlockSpec((1, BG, 2, TL), lambda d, t: (d, 0, 0, t)),
            pl.BlockSpec((BG, 2), lambda d, t: (0, 0)),
        ],
        out_shape=[
            jax.ShapeDtypeStruct((D_DEPTH, BG, 2, L_WIDTH), jnp.float32),
            jax.ShapeDtypeStruct((BG, 2), jnp.float32),
        ],
        scratch_shapes=[pltpu.VMEM((2, BG, L_WIDTH, C), jnp.float32)],
    )(emb2, table2, w1pbd, w1ebd, b1p, w2bd, b2p, woutm2, boutm,
      woutr2, boutr)
    return outm, outr


def _dag_body(emb_ref, table_ref, w1p_ref, w1e_ref, b1_ref, w2_ref, b2_ref,
               woutm_ref, boutm_ref, woutr_ref, boutr_ref,
               outm_ref, outr_ref, slab_ref):
    d = pl.program_id(0)
    t = pl.program_id(1)

    sel = (jax.lax.broadcasted_iota(jnp.int32, (2, C), 1) // H ==
           jax.lax.broadcasted_iota(jnp.int32, (2, C), 0)).astype(jnp.float32)

    @pl.when(jnp.logical_and(d == 0, t == 0))
    def _root_out():
        prod = emb_ref[:] * woutr_ref[:]
        outr_ref[:] = jax.lax.dot_general(
            prod, sel,
            dimension_numbers=(((1,), (1,)), ((), ())),
            preferred_element_type=jnp.float32) + boutr_ref[:]

    r = (d + 1) % 2
    w = d % 2
    base = t * TL
    nxt = ((t + 1) % NT) * TL

    @pl.when(d == 0)
    def _seed():
        seed = jnp.broadcast_to((emb_ref[:] * 0.0625)[:, None, :],
                                (BG, TL, C))
        slab_ref[r, :, pl.ds(base, TL), :] = seed
        slab_ref[r, :, pl.ds(nxt, 16), :] = seed[:, :16, :]

    x = slab_ref[r, :, pl.ds(base, TL), :]
    x16 = slab_ref[r, :, pl.ds(nxt, 16), :]
    s = jnp.concatenate([x, x16], axis=1)
    ri = jax.lax.broadcasted_iota(jnp.int32, (128, 144), 0)
    ci = jax.lax.broadcasted_iota(jnp.int32, (128, 144), 1)
    band = jnp.logical_and(ci >= ri, ci < ri + 16).astype(jnp.float32)
    bandb = jnp.broadcast_to(band[None], (BG, 128, 144))
    parts = []
    for u in range(TL // 128):
        su = s[:, 128 * u:128 * u + 144, :]
        parts.append(jax.lax.dot_general(
            bandb, su,
            dimension_numbers=(((2,), (1,)), ((0,), (0,))),
            preferred_element_type=jnp.float32))
    pv = jnp.concatenate(parts, axis=1)

    np_tile = jax.lax.dot_general(
        table_ref[:], w1e_ref[:],
        dimension_numbers=(((1,), (1,)), ((), ())),
        preferred_element_type=jnp.float32) + b1_ref[:]
    pv2 = pv.reshape(BG * TL, C)
    h1 = jax.lax.dot_general(
        pv2, w1p_ref[:],
        dimension_numbers=(((1,), (1,)), ((), ())),
        preferred_element_type=jnp.float32).reshape(BG, TL, C)
    h1 = h1 + np_tile[None]
    a = jnp.maximum(h1, 0.0).reshape(BG * TL, C)
    h2 = jax.lax.dot_general(
        a, w2_ref[:],
        dimension_numbers=(((1,), (1,)), ((), ())),
        preferred_element_type=jnp.float32).reshape(BG, TL, C)
    cur = pv + h2 + b2_ref[:][None]

    slab_ref[w, :, pl.ds(base, TL), :] = cur
    p = cur * woutm_ref[:][None]
    red = jax.lax.dot_general(
        jnp.broadcast_to(sel[None], (BG, 2, C)), p,
        dimension_numbers=(((2,), (2,)), ((0,), (0,))),
        preferred_element_type=jnp.float32)
    outm_ref[0] = red + boutm_ref[0, 0][None, None, :]
